# Initial kernel scaffold; baseline (speedup 1.0000x reference)
import jax
import jax.numpy as jnp
from jax.experimental import pallas as pl


def kernel(user_emb, item_emb, user_prototypes, item_prototypes, adj_indices, adj_values):
    raise NotImplementedError("write your pallas kernel here")


# speedup vs baseline: 2.9110x; 2.9110x over previous
import functools

import jax
import jax.numpy as jnp
from jax import lax
from jax.experimental import pallas as pl
from jax.experimental.pallas import tpu as pltpu
from jax.experimental.pallas import tpu_sc as plsc

_USER_NUM = 4000
_ITEM_NUM = 6000
_N = _USER_NUM + _ITEM_NUM
_NP = 10240
_D = 128
_HD = _D // 2
_E = 320000

_NC = 2
_NS = 16
_CH = 128
_NCHUNK = -(-_E // (_NS * _CH))
_EPT = _NCHUNK * _CH
_EPAD = _NS * _EPT

_RPT = _NP // _NS
_RCH = 128


@functools.partial(
    pl.kernel,
    out_type=jax.ShapeDtypeStruct((_NC, _NP, _HD), jnp.float32),
    mesh=plsc.VectorSubcoreMesh(core_axis_name="c", subcore_axis_name="s"),
    compiler_params=pltpu.CompilerParams(use_tc_tiling_on_sc=False),
    scratch_types=[
        pltpu.VMEM((_NCHUNK, _CH), jnp.int32),
        pltpu.VMEM((_NCHUNK, _CH), jnp.int32),
        pltpu.VMEM((_CH, 16), jnp.float32),
        pltpu.VMEM((_CH, _HD), jnp.float32),
        pltpu.VMEM_SHARED((_NP, _HD), jnp.float32),
        pltpu.SemaphoreType.DMA,
    ],
)
def _sc_layer(src, cols, rows, vals, out, cols_v, rows_v, vals_v, buf_v, acc_sh, sem):
    c = lax.axis_index("c")
    s = lax.axis_index("s")

    zero16 = jnp.zeros((16,), jnp.float32)

    def _zb(e, carry):
        for k in range(_HD // 16):
            buf_v[e, pl.ds(k * 16, 16)] = zero16
        return carry

    lax.fori_loop(0, _CH, _zb, 0)

    r0 = s * _RPT
    for i in range(_RPT // _RCH):
        pltpu.sync_copy(buf_v.at[pl.ds(0, _RCH)],
                        acc_sh.at[pl.ds(r0 + i * _RCH, _RCH)])

    pltpu.sync_copy(cols.at[s], cols_v)
    pltpu.sync_copy(rows.at[s], rows_v)

    plsc.subcore_barrier()

    def _chunk(j, carry):
        pltpu.sync_copy(vals.at[s, j], vals_v)
        pltpu.async_copy(src.at[c].at[cols_v.at[j]], buf_v, sem).wait()

        def _scale(e, c2):
            v = vals_v[e]
            for k in range(_HD // 16):
                sl = pl.ds(k * 16, 16)
                buf_v[e, sl] = buf_v[e, sl] * v
            return c2

        lax.fori_loop(0, _CH, _scale, 0)
        pltpu.sync_copy(buf_v, acc_sh.at[rows_v.at[j]], add=True)
        return carry

    lax.fori_loop(0, _NCHUNK, _chunk, 0)

    plsc.subcore_barrier()

    for i in range(_RPT // _RCH):
        sl = pl.ds(r0 + i * _RCH, _RCH)
        pltpu.sync_copy(acc_sh.at[sl], buf_v.at[pl.ds(0, _RCH)])
        pltpu.sync_copy(buf_v.at[pl.ds(0, _RCH)], out.at[c, sl])


def _mean_body(e0, e1, e2, e3, o):
    o[...] = (e0[...] + e1[...] + e2[...] + e3[...]) * 0.25


def _mean(e0, e1, e2, e3):
    spec = pl.BlockSpec((2000, _D), lambda i: (i, 0))
    return pl.pallas_call(
        _mean_body,
        out_shape=jax.ShapeDtypeStruct((_N, _D), jnp.float32),
        grid=(5,),
        in_specs=[spec] * 4,
        out_specs=spec,
    )(e0, e1, e2, e3)


def _unsplit(t):
    return t.transpose(1, 0, 2).reshape(_NP, _D)[:_N]


def kernel(user_emb, item_emb, user_prototypes, item_prototypes, adj_indices, adj_values):
    e0 = jnp.concatenate([user_emb, item_emb], axis=0)
    e0s = jnp.pad(e0, ((0, _NP - _N), (0, 0))).reshape(_NP, _NC, _HD).transpose(1, 0, 2)

    pad = _EPAD - _E
    rows = jnp.pad(adj_indices[0], (0, pad)).reshape(_NS, _NCHUNK, _CH)
    cols = jnp.pad(adj_indices[1], (0, pad)).reshape(_NS, _NCHUNK, _CH)
    vals = jnp.broadcast_to(
        jnp.pad(adj_values, (0, pad)).reshape(_NS, _NCHUNK, _CH)[..., None],
        (_NS, _NCHUNK, _CH, 16))

    t1 = _sc_layer(e0s, cols, rows, vals)
    t2 = _sc_layer(t1, cols, rows, vals)
    t3 = _sc_layer(t2, cols, rows, vals)
    e1 = _unsplit(t1)
    e2 = _unsplit(t2)
    e3 = _unsplit(t3)
    mean = _mean(e0, e1, e2, e3)

    return (mean[:_USER_NUM], mean[_USER_NUM:], user_prototypes,
            item_prototypes, (e0, e1, e2, e3))


# speedup vs baseline: 3.7433x; 1.2859x over previous
import functools

import jax
import jax.numpy as jnp
from jax import lax
from jax.experimental import pallas as pl
from jax.experimental.pallas import tpu as pltpu
from jax.experimental.pallas import tpu_sc as plsc

_USER_NUM = 4000
_ITEM_NUM = 6000
_N = _USER_NUM + _ITEM_NUM
_NP = 10240
_D = 128
_HD = _D // 2
_E = 320000

_NC = 2
_NS = 16
_CH = 128
_NCHUNK = 158
_EPT = _NCHUNK * _CH
_EPAD = _NS * _EPT

_RPT = _NP // _NS
_RCH = 128


@functools.partial(
    pl.kernel,
    out_type=jax.ShapeDtypeStruct((_NC, _NP, _HD), jnp.float32),
    mesh=plsc.VectorSubcoreMesh(core_axis_name="c", subcore_axis_name="s"),
    compiler_params=pltpu.CompilerParams(use_tc_tiling_on_sc=False),
    scratch_types=[
        pltpu.VMEM((_NCHUNK, _CH), jnp.int32),
        pltpu.VMEM((_NCHUNK, _CH), jnp.int32),
        pltpu.VMEM((_CH, 16), jnp.float32),
        pltpu.VMEM((_CH, _HD), jnp.float32),
        pltpu.VMEM((_CH, _HD), jnp.float32),
        pltpu.VMEM_SHARED((_NP, _HD), jnp.float32),
        pltpu.SemaphoreType.DMA,
        pltpu.SemaphoreType.DMA,
    ],
)
def _sc_layer(src, cols, rows, vals, out, cols_v, rows_v, vals_v, buf_v, buf_w, acc_sh, sem0, sem1):
    c = lax.axis_index("c")
    s = lax.axis_index("s")

    zero16 = jnp.zeros((16,), jnp.float32)

    def _zb(e, carry):
        for k in range(_HD // 16):
            buf_v[e, pl.ds(k * 16, 16)] = zero16
        return carry

    lax.fori_loop(0, _CH, _zb, 0)

    r0 = s * _RPT
    for i in range(_RPT // _RCH):
        pltpu.sync_copy(buf_v.at[pl.ds(0, _RCH)],
                        acc_sh.at[pl.ds(r0 + i * _RCH, _RCH)])

    pltpu.sync_copy(cols.at[s], cols_v)
    pltpu.sync_copy(rows.at[s], rows_v)
    pltpu.async_copy(src.at[c].at[cols_v.at[0]], buf_v, sem0)
    pltpu.async_copy(src.at[c].at[cols_v.at[1]], buf_w, sem1)

    plsc.subcore_barrier()

    def _process(j, jn, buf, sem):
        pltpu.make_async_copy(src.at[c].at[cols_v.at[j]], buf, sem).wait()
        pltpu.sync_copy(vals.at[s, j], vals_v)

        def _scale(e, c2):
            v = vals_v[e]
            for k in range(_HD // 16):
                sl = pl.ds(k * 16, 16)
                buf[e, sl] = buf[e, sl] * v
            return c2

        lax.fori_loop(0, _CH, _scale, 0, unroll=4)
        pltpu.sync_copy(buf, acc_sh.at[rows_v.at[j]], add=True)

        @pl.when(jn < _NCHUNK)
        def _():
            pltpu.async_copy(src.at[c].at[cols_v.at[jn]], buf, sem)

    def _pair(p, carry):
        j0 = 2 * p
        _process(j0, j0 + 2, buf_v, sem0)
        _process(j0 + 1, j0 + 3, buf_w, sem1)
        return carry

    lax.fori_loop(0, _NCHUNK // 2, _pair, 0)

    plsc.subcore_barrier()

    for i in range(_RPT // _RCH):
        sl = pl.ds(r0 + i * _RCH, _RCH)
        pltpu.sync_copy(acc_sh.at[sl], buf_v.at[pl.ds(0, _RCH)])
        pltpu.sync_copy(buf_v.at[pl.ds(0, _RCH)], out.at[c, sl])


def _mean_body(e0, e1, e2, e3, o):
    o[...] = (e0[...] + e1[...] + e2[...] + e3[...]) * 0.25


def _mean(e0, e1, e2, e3):
    spec = pl.BlockSpec((2000, _D), lambda i: (i, 0))
    return pl.pallas_call(
        _mean_body,
        out_shape=jax.ShapeDtypeStruct((_N, _D), jnp.float32),
        grid=(5,),
        in_specs=[spec] * 4,
        out_specs=spec,
    )(e0, e1, e2, e3)


def _unsplit(t):
    return t.transpose(1, 0, 2).reshape(_NP, _D)[:_N]


def kernel(user_emb, item_emb, user_prototypes, item_prototypes, adj_indices, adj_values):
    e0 = jnp.concatenate([user_emb, item_emb], axis=0)
    e0s = jnp.pad(e0, ((0, _NP - _N), (0, 0))).reshape(_NP, _NC, _HD).transpose(1, 0, 2)

    pad = _EPAD - _E
    rows = jnp.pad(adj_indices[0], (0, pad)).reshape(_NS, _NCHUNK, _CH)
    cols = jnp.pad(adj_indices[1], (0, pad)).reshape(_NS, _NCHUNK, _CH)
    vals = jnp.broadcast_to(
        jnp.pad(adj_values, (0, pad)).reshape(_NS, _NCHUNK, _CH)[..., None],
        (_NS, _NCHUNK, _CH, 16))

    t1 = _sc_layer(e0s, cols, rows, vals)
    t2 = _sc_layer(t1, cols, rows, vals)
    t3 = _sc_layer(t2, cols, rows, vals)
    e1 = _unsplit(t1)
    e2 = _unsplit(t2)
    e3 = _unsplit(t3)
    mean = _mean(e0, e1, e2, e3)

    return (mean[:_USER_NUM], mean[_USER_NUM:], user_prototypes,
            item_prototypes, (e0, e1, e2, e3))


# speedup vs baseline: 4.0793x; 1.0898x over previous
import functools

import jax
import jax.numpy as jnp
from jax import lax
from jax.experimental import pallas as pl
from jax.experimental.pallas import tpu as pltpu
from jax.experimental.pallas import tpu_sc as plsc

_USER_NUM = 4000
_ITEM_NUM = 6000
_N = _USER_NUM + _ITEM_NUM
_NP = 10240
_D = 128
_HD = _D // 2
_E = 320000

_NC = 2
_NS = 16
_CH = 128
_NBUF = 4
_NCHUNK = 160
_EPT = _NCHUNK * _CH
_EPAD = _NS * _EPT

_RPT = _NP // _NS
_RCH = 128


@functools.partial(
    pl.kernel,
    out_type=jax.ShapeDtypeStruct((_NC, _NP, _HD), jnp.float32),
    mesh=plsc.VectorSubcoreMesh(core_axis_name="c", subcore_axis_name="s"),
    compiler_params=pltpu.CompilerParams(use_tc_tiling_on_sc=False),
    scratch_types=(
        [pltpu.VMEM((_NCHUNK, _CH), jnp.int32)] * 2
        + [pltpu.VMEM((_CH, _HD), jnp.float32)] * _NBUF
        + [pltpu.VMEM((_CH, 16), jnp.float32)] * _NBUF
        + [pltpu.VMEM_SHARED((_NP, _HD), jnp.float32)]
        + [pltpu.SemaphoreType.DMA] * (3 * _NBUF)
    ),
)
def _sc_layer(src, cols, rows, vals, out, cols_v, rows_v, *scr):
    gbuf = scr[:_NBUF]
    vbuf = scr[_NBUF:2 * _NBUF]
    acc_sh = scr[2 * _NBUF]
    gsem = scr[2 * _NBUF + 1:2 * _NBUF + 1 + _NBUF]
    vsem = scr[2 * _NBUF + 1 + _NBUF:2 * _NBUF + 1 + 2 * _NBUF]
    ssem = scr[2 * _NBUF + 1 + 2 * _NBUF:]

    c = lax.axis_index("c")
    s = lax.axis_index("s")

    zero16 = jnp.zeros((16,), jnp.float32)

    def _zb(e, carry):
        for k in range(_HD // 16):
            gbuf[0][e, pl.ds(k * 16, 16)] = zero16
        return carry

    lax.fori_loop(0, _CH, _zb, 0)

    r0 = s * _RPT
    for i in range(_RPT // _RCH):
        pltpu.sync_copy(gbuf[0].at[pl.ds(0, _RCH)],
                        acc_sh.at[pl.ds(r0 + i * _RCH, _RCH)])

    pltpu.sync_copy(cols.at[s], cols_v)
    pltpu.sync_copy(rows.at[s], rows_v)
    for b in range(2):
        pltpu.async_copy(src.at[c].at[cols_v.at[b]], gbuf[b], gsem[b])
        pltpu.async_copy(vals.at[s, b], vbuf[b], vsem[b])

    plsc.subcore_barrier()

    def _group(g, carry):
        j0 = g * _NBUF
        for b in range(_NBUF):
            j = j0 + b
            bn = (b + 2) % _NBUF
            pltpu.make_async_copy(src.at[c].at[cols_v.at[j]], gbuf[b], gsem[b]).wait()
            pltpu.make_async_copy(vals.at[s, j], vbuf[b], vsem[b]).wait()

            def _scale(e, c2, _b=b):
                v = vbuf[_b][e]
                for k in range(_HD // 16):
                    sl = pl.ds(k * 16, 16)
                    gbuf[_b][e, sl] = gbuf[_b][e, sl] * v
                return c2

            lax.fori_loop(0, _CH, _scale, 0, unroll=4)
            pltpu.async_copy(gbuf[b], acc_sh.at[rows_v.at[j]], ssem[b], add=True)

            jn = j + 2

            @pl.when(jn < _NCHUNK)
            def _(j=j, jn=jn, b=b, bn=bn):
                @pl.when(j >= 2)
                def _():
                    pltpu.make_async_copy(gbuf[bn], acc_sh.at[rows_v.at[j]],
                                          ssem[bn]).wait()

                pltpu.async_copy(src.at[c].at[cols_v.at[jn]], gbuf[bn], gsem[bn])
                pltpu.async_copy(vals.at[s, jn], vbuf[bn], vsem[bn])

        return carry

    lax.fori_loop(0, _NCHUNK // _NBUF, _group, 0)

    for b in range(_NBUF):
        pltpu.make_async_copy(gbuf[b], acc_sh.at[rows_v.at[0]], ssem[b]).wait()

    plsc.subcore_barrier()

    for i in range(_RPT // _RCH):
        sl = pl.ds(r0 + i * _RCH, _RCH)
        pltpu.sync_copy(acc_sh.at[sl], gbuf[0].at[pl.ds(0, _RCH)])
        pltpu.sync_copy(gbuf[0].at[pl.ds(0, _RCH)], out.at[c, sl])


def _mean_body(e0, e1, e2, e3, o):
    o[...] = (e0[...] + e1[...] + e2[...] + e3[...]) * 0.25


def _mean(e0, e1, e2, e3):
    spec = pl.BlockSpec((2000, _D), lambda i: (i, 0))
    return pl.pallas_call(
        _mean_body,
        out_shape=jax.ShapeDtypeStruct((_N, _D), jnp.float32),
        grid=(5,),
        in_specs=[spec] * 4,
        out_specs=spec,
    )(e0, e1, e2, e3)


def _unsplit(t):
    return t.transpose(1, 0, 2).reshape(_NP, _D)[:_N]


def kernel(user_emb, item_emb, user_prototypes, item_prototypes, adj_indices, adj_values):
    e0 = jnp.concatenate([user_emb, item_emb], axis=0)
    e0s = jnp.pad(e0, ((0, _NP - _N), (0, 0))).reshape(_NP, _NC, _HD).transpose(1, 0, 2)

    pad = _EPAD - _E
    rows = jnp.pad(adj_indices[0], (0, pad)).reshape(_NS, _NCHUNK, _CH)
    cols = jnp.pad(adj_indices[1], (0, pad)).reshape(_NS, _NCHUNK, _CH)
    vals = jnp.broadcast_to(
        jnp.pad(adj_values, (0, pad)).reshape(_NS, _NCHUNK, _CH)[..., None],
        (_NS, _NCHUNK, _CH, 16))

    t1 = _sc_layer(e0s, cols, rows, vals)
    t2 = _sc_layer(t1, cols, rows, vals)
    t3 = _sc_layer(t2, cols, rows, vals)
    e1 = _unsplit(t1)
    e2 = _unsplit(t2)
    e3 = _unsplit(t3)
    mean = _mean(e0, e1, e2, e3)

    return (mean[:_USER_NUM], mean[_USER_NUM:], user_prototypes,
            item_prototypes, (e0, e1, e2, e3))
